# baseline (speedup 1.0000x reference)
import functools
import numpy as np
import jax
from jax import lax
import jax.numpy as jnp
from jax.experimental import pallas as pl
from jax.experimental.pallas import tpu as pltpu
from jax.experimental.pallas import tpu_sc as plsc

_N, _CH, _H, _W = 10, 64, 128, 256
_L = 5
_NW = 32
_BAND = 8
_TASKS = _N * (_H // _BAND)
_TPW = _TASKS // _NW
_PADW = 288

_xg, _yg = np.mgrid[-2:3, -2:3]
_GK = (1.0 / (2.0 * np.pi) * np.exp(-(np.square(_xg) + np.square(_yg)) / 2.0)).astype(np.float32)
_GK_BF = _GK.astype(jnp.bfloat16).astype(np.float32)


def _lane_rot(v, idx):
    dnums = lax.GatherDimensionNumbers(
        offset_dims=(), collapsed_slice_dims=(0,), start_index_map=(0,))
    return lax.gather(v, idx[:, None], dnums, (1,),
                      mode=lax.GatherScatterMode.PROMISE_IN_BOUNDS)


def _bf16_round(x):
    u = lax.bitcast_convert_type(x, jnp.int32)
    r = (u + 0x7FFF + ((u >> 16) & 1)) & jnp.int32(-65536)
    return lax.bitcast_convert_type(r, jnp.float32)


def _sc_mask_body(conf_hbm, masks_hbm, parts_hbm, in0, in1, padb, shb, outb, acc0, acc1):
    cid = lax.axis_index("c")
    sid = lax.axis_index("s")
    wid = cid * 16 + sid
    zeros16 = jnp.zeros((16,), jnp.float32)
    lane = lax.iota(jnp.int32, 16)
    rot14 = (lane + 14) & 15
    rot15 = (lane + 15) & 15
    rot1 = (lane + 1) & 15
    rot2 = (lane + 2) & 15

    acc0[...] = zeros16
    acc1[...] = zeros16
    for h in range(12):
        for w in range(0, _PADW, 16):
            padb[h, pl.ds(w, 16)] = zeros16

    def _task(k, carry):
        t = wid + _NW * k

        @pl.when(t < _TASKS)
        def _do_task():
            n = t // (_H // _BAND)
            r0 = (t % (_H // _BAND)) * _BAND
            top = r0 == 0
            bot = r0 == (_H - _BAND)

            @pl.when(top)
            def _stage_top():
                pltpu.sync_copy(conf_hbm.at[n, 0, pl.ds(0, 16)], in0.at[pl.ds(8, 16)])
                pltpu.sync_copy(conf_hbm.at[n, 1, pl.ds(0, 16)], in1.at[pl.ds(8, 16)])

            @pl.when(bot)
            def _stage_bot():
                pltpu.sync_copy(conf_hbm.at[n, 0, pl.ds(_H - 16, 16)], in0.at[pl.ds(0, 16)])
                pltpu.sync_copy(conf_hbm.at[n, 1, pl.ds(_H - 16, 16)], in1.at[pl.ds(0, 16)])

            @pl.when(jnp.logical_not(top | bot))
            def _stage_mid():
                pltpu.sync_copy(conf_hbm.at[n, 0, pl.ds(r0 - 8, 24)], in0)
                pltpu.sync_copy(conf_hbm.at[n, 1, pl.ds(r0 - 8, 24)], in1)

            def _row(h):
                edge = (top & (h < 2)) | (bot & (h >= 10))
                for wi in range(_W // 16):
                    w = wi * 16
                    v = jnp.maximum(in0[h + 6, pl.ds(w, 16)], in1[h + 6, pl.ds(w, 16)])
                    s = 1.0 / (1.0 + jnp.exp(-v))
                    padb[h, pl.ds(w + 16, 16)] = jnp.where(edge, zeros16, _bf16_round(s))
                for wi in range(_W // 16):
                    w0 = wi * 16
                    a = padb[h, pl.ds(w0, 16)]
                    b = padb[h, pl.ds(w0 + 16, 16)]
                    c = padb[h, pl.ds(w0 + 32, 16)]
                    shb[0, h, pl.ds(w0, 16)] = jnp.where(
                        lane < 2, _lane_rot(a, rot14), _lane_rot(b, rot14))
                    shb[1, h, pl.ds(w0, 16)] = jnp.where(
                        lane < 1, _lane_rot(a, rot15), _lane_rot(b, rot15))
                    shb[2, h, pl.ds(w0, 16)] = jnp.where(
                        lane < 15, _lane_rot(b, rot1), _lane_rot(c, rot1))
                    shb[3, h, pl.ds(w0, 16)] = jnp.where(
                        lane < 14, _lane_rot(b, rot2), _lane_rot(c, rot2))

            plsc.parallel_loop(0, 12, unroll=2)(_row)

            def _conv_row(h, cnt):
                for wi in range(_W // 16):
                    w0 = wi * 16
                    rows = []
                    for i in range(5):
                        acc_i = _GK_BF[i, 0] * shb[0, h + i, pl.ds(w0, 16)]
                        acc_i = acc_i + _GK_BF[i, 1] * shb[1, h + i, pl.ds(w0, 16)]
                        acc_i = acc_i + _GK_BF[i, 2] * padb[h + i, pl.ds(w0 + 16, 16)]
                        acc_i = acc_i + _GK_BF[i, 3] * shb[2, h + i, pl.ds(w0, 16)]
                        acc_i = acc_i + _GK_BF[i, 4] * shb[3, h + i, pl.ds(w0, 16)]
                        rows.append(acc_i)
                    acc = ((rows[0] + rows[1]) + (rows[2] + rows[3])) + rows[4]
                    th = jnp.where(acc > 0.5, 1.0, 0.0)
                    outb[h, pl.ds(w0, 16)] = th
                    cnt = cnt + th
                return cnt

            cntv = plsc.parallel_loop(0, _BAND, unroll=2, carry=zeros16)(_conv_row)

            @pl.when(n < _L)
            def _acc_b0():
                acc0[...] = acc0[...] + cntv

            @pl.when(n >= _L)
            def _acc_b1():
                acc1[...] = acc1[...] + cntv

            pltpu.sync_copy(outb, masks_hbm.at[n, pl.ds(r0, _BAND)])

        return carry

    lax.fori_loop(0, _TPW, _task, 0)
    pltpu.sync_copy(acc0, parts_hbm.at[wid, 0])
    pltpu.sync_copy(acc1, parts_hbm.at[wid, 1])


def _apply_body(parts_ref, x_ref, mask_ref, out_ref, counts_ref):
    n = pl.program_id(0)

    @pl.when(n == 0)
    def _reduce_counts():
        counts_ref[0, 0] = jnp.sum(parts_ref[0:4])
        counts_ref[0, 1] = jnp.sum(parts_ref[4:8])

    mv = mask_ref[0]
    m_eff = jnp.where(n % _L == 0, jnp.ones_like(mv), mv)
    prod = x_ref[0] * m_eff[None]
    out_ref[0, :_CH] = prod
    out_ref[0, _CH:] = prod


def kernel(x, batch_confidence_maps, batch_rm_sigle, batch_targets_label, B, gk):
    conf = batch_confidence_maps.reshape(_N, 2, _H, _W)

    mesh = plsc.VectorSubcoreMesh(core_axis_name="c", subcore_axis_name="s")
    sc_mask = functools.partial(
        pl.kernel,
        mesh=mesh,
        out_type=[
            jax.ShapeDtypeStruct((_N, _H, _W), jnp.float32),
            jax.ShapeDtypeStruct((_NW, 2, 16), jnp.float32),
        ],
        scratch_types=[
            pltpu.VMEM((24, _W), jnp.float32),
            pltpu.VMEM((24, _W), jnp.float32),
            pltpu.VMEM((12, _PADW), jnp.float32),
            pltpu.VMEM((4, 12, _W), jnp.float32),
            pltpu.VMEM((_BAND, _W), jnp.float32),
            pltpu.VMEM((16,), jnp.float32),
            pltpu.VMEM((16,), jnp.float32),
        ],
    )(_sc_mask_body)
    masks, parts = sc_mask(conf)

    xo, counts = pl.pallas_call(
        _apply_body,
        grid=(_N,),
        in_specs=[
            pl.BlockSpec((8, 128), lambda n: (0, 0)),
            pl.BlockSpec((1, _CH, _H, _W), lambda n: (n, 0, 0, 0)),
            pl.BlockSpec((1, _H, _W), lambda n: (n, 0, 0)),
        ],
        out_specs=[
            pl.BlockSpec((1, 2 * _CH, _H, _W), lambda n: (n, 0, 0, 0)),
            pl.BlockSpec((1, 2), lambda n: (0, 0), memory_space=pltpu.SMEM),
        ],
        out_shape=[
            jax.ShapeDtypeStruct((_N, 2 * _CH, _H, _W), jnp.float32),
            jax.ShapeDtypeStruct((1, 2), jnp.float32),
        ],
    )(parts.reshape(8, 128), x, masks)

    denom = jnp.float32(_L * _H * _W)
    rate = (counts[0, 0] / denom + counts[0, 1] / denom) / 2
    return xo, rate


# speedup vs baseline: 1.0759x; 1.0759x over previous
import functools
import numpy as np
import jax
from jax import lax
import jax.numpy as jnp
from jax.experimental import pallas as pl
from jax.experimental.pallas import tpu as pltpu
from jax.experimental.pallas import tpu_sc as plsc

_N, _CH, _H, _W = 10, 64, 128, 256
_L = 5
_NW = 32
_BAND = 8
_TASKS = _N * (_H // _BAND)
_TPW = _TASKS // _NW
_PADW = 288

_xg, _yg = np.mgrid[-2:3, -2:3]
_GK = (1.0 / (2.0 * np.pi) * np.exp(-(np.square(_xg) + np.square(_yg)) / 2.0)).astype(np.float32)
_GK_BF = _GK.astype(jnp.bfloat16).astype(np.float32)


def _lane_rot(v, idx):
    dnums = lax.GatherDimensionNumbers(
        offset_dims=(), collapsed_slice_dims=(0,), start_index_map=(0,))
    return lax.gather(v, idx[:, None], dnums, (1,),
                      mode=lax.GatherScatterMode.PROMISE_IN_BOUNDS)


def _bf16_round(x):
    u = lax.bitcast_convert_type(x, jnp.int32)
    r = (u + 0x7FFF + ((u >> 16) & 1)) & jnp.int32(-65536)
    return lax.bitcast_convert_type(r, jnp.float32)


def _sc_mask_body(conf_hbm, masks_hbm, parts_hbm, in0, in1, padb, shb, outb, acc0, acc1):
    cid = lax.axis_index("c")
    sid = lax.axis_index("s")
    wid = cid * 16 + sid
    zeros16 = jnp.zeros((16,), jnp.float32)
    lane = lax.iota(jnp.int32, 16)
    rot14 = (lane + 14) & 15
    rot15 = (lane + 15) & 15
    rot1 = (lane + 1) & 15
    rot2 = (lane + 2) & 15

    acc0[...] = zeros16
    acc1[...] = zeros16
    for h in range(12):
        for w in range(0, _PADW, 16):
            padb[h, pl.ds(w, 16)] = zeros16

    def _task(k, carry):
        t = wid + _NW * k

        @pl.when(t < _TASKS)
        def _do_task():
            n = t // (_H // _BAND)
            r0 = (t % (_H // _BAND)) * _BAND
            top = r0 == 0
            bot = r0 == (_H - _BAND)

            @pl.when(top)
            def _stage_top():
                pltpu.sync_copy(conf_hbm.at[n, 0, pl.ds(0, 16)], in0.at[pl.ds(8, 16)])
                pltpu.sync_copy(conf_hbm.at[n, 1, pl.ds(0, 16)], in1.at[pl.ds(8, 16)])

            @pl.when(bot)
            def _stage_bot():
                pltpu.sync_copy(conf_hbm.at[n, 0, pl.ds(_H - 16, 16)], in0.at[pl.ds(0, 16)])
                pltpu.sync_copy(conf_hbm.at[n, 1, pl.ds(_H - 16, 16)], in1.at[pl.ds(0, 16)])

            @pl.when(jnp.logical_not(top | bot))
            def _stage_mid():
                pltpu.sync_copy(conf_hbm.at[n, 0, pl.ds(r0 - 8, 24)], in0)
                pltpu.sync_copy(conf_hbm.at[n, 1, pl.ds(r0 - 8, 24)], in1)

            def _row(h):
                edge = (top & (h < 2)) | (bot & (h >= 10))
                for wi in range(_W // 16):
                    w = wi * 16
                    v = jnp.maximum(in0[h + 6, pl.ds(w, 16)], in1[h + 6, pl.ds(w, 16)])
                    s = 1.0 / (1.0 + jnp.exp(-v))
                    padb[h, pl.ds(w + 16, 16)] = jnp.where(edge, zeros16, _bf16_round(s))
                for wi in range(_W // 16):
                    w0 = wi * 16
                    a = padb[h, pl.ds(w0, 16)]
                    b = padb[h, pl.ds(w0 + 16, 16)]
                    c = padb[h, pl.ds(w0 + 32, 16)]
                    shb[0, h, pl.ds(w0, 16)] = jnp.where(
                        lane < 2, _lane_rot(a, rot14), _lane_rot(b, rot14))
                    shb[1, h, pl.ds(w0, 16)] = jnp.where(
                        lane < 1, _lane_rot(a, rot15), _lane_rot(b, rot15))
                    shb[2, h, pl.ds(w0, 16)] = jnp.where(
                        lane < 15, _lane_rot(b, rot1), _lane_rot(c, rot1))
                    shb[3, h, pl.ds(w0, 16)] = jnp.where(
                        lane < 14, _lane_rot(b, rot2), _lane_rot(c, rot2))

            plsc.parallel_loop(0, 12)(_row)

            def _conv_row(h, cnt):
                for wi in range(_W // 16):
                    w0 = wi * 16
                    rows = []
                    for i in range(5):
                        acc_i = _GK_BF[i, 0] * shb[0, h + i, pl.ds(w0, 16)]
                        acc_i = acc_i + _GK_BF[i, 1] * shb[1, h + i, pl.ds(w0, 16)]
                        acc_i = acc_i + _GK_BF[i, 2] * padb[h + i, pl.ds(w0 + 16, 16)]
                        acc_i = acc_i + _GK_BF[i, 3] * shb[2, h + i, pl.ds(w0, 16)]
                        acc_i = acc_i + _GK_BF[i, 4] * shb[3, h + i, pl.ds(w0, 16)]
                        rows.append(acc_i)
                    acc = ((rows[0] + rows[1]) + (rows[2] + rows[3])) + rows[4]
                    th = jnp.where(acc > 0.5, 1.0, 0.0)
                    outb[h, pl.ds(w0, 16)] = th
                    cnt = cnt + th
                return cnt

            cntv = plsc.parallel_loop(0, _BAND, carry=zeros16)(_conv_row)

            @pl.when(n < _L)
            def _acc_b0():
                acc0[...] = acc0[...] + cntv

            @pl.when(n >= _L)
            def _acc_b1():
                acc1[...] = acc1[...] + cntv

            pltpu.sync_copy(outb, masks_hbm.at[n, pl.ds(r0, _BAND)])

        return carry

    lax.fori_loop(0, _TPW, _task, 0)
    pltpu.sync_copy(acc0, parts_hbm.at[wid, 0])
    pltpu.sync_copy(acc1, parts_hbm.at[wid, 1])


def _apply_body(parts_ref, x_ref, mask_ref, out_ref, counts_ref):
    n = pl.program_id(0)

    @pl.when(n == 0)
    def _reduce_counts():
        counts_ref[0, 0] = jnp.sum(parts_ref[0:4])
        counts_ref[0, 1] = jnp.sum(parts_ref[4:8])

    mv = mask_ref[0]
    m_eff = jnp.where(n % _L == 0, jnp.ones_like(mv), mv)
    prod = x_ref[0] * m_eff[None]
    out_ref[0, :_CH] = prod
    out_ref[0, _CH:] = prod


def kernel(x, batch_confidence_maps, batch_rm_sigle, batch_targets_label, B, gk):
    conf = batch_confidence_maps.reshape(_N, 2, _H, _W)

    mesh = plsc.VectorSubcoreMesh(core_axis_name="c", subcore_axis_name="s")
    sc_mask = functools.partial(
        pl.kernel,
        mesh=mesh,
        out_type=[
            jax.ShapeDtypeStruct((_N, _H, _W), jnp.float32),
            jax.ShapeDtypeStruct((_NW, 2, 16), jnp.float32),
        ],
        scratch_types=[
            pltpu.VMEM((24, _W), jnp.float32),
            pltpu.VMEM((24, _W), jnp.float32),
            pltpu.VMEM((12, _PADW), jnp.float32),
            pltpu.VMEM((4, 12, _W), jnp.float32),
            pltpu.VMEM((_BAND, _W), jnp.float32),
            pltpu.VMEM((16,), jnp.float32),
            pltpu.VMEM((16,), jnp.float32),
        ],
    )(_sc_mask_body)
    masks, parts = sc_mask(conf)

    xo, counts = pl.pallas_call(
        _apply_body,
        grid=(_N,),
        in_specs=[
            pl.BlockSpec((8, 128), lambda n: (0, 0)),
            pl.BlockSpec((1, _CH, _H, _W), lambda n: (n, 0, 0, 0)),
            pl.BlockSpec((1, _H, _W), lambda n: (n, 0, 0)),
        ],
        out_specs=[
            pl.BlockSpec((1, 2 * _CH, _H, _W), lambda n: (n, 0, 0, 0)),
            pl.BlockSpec((1, 2), lambda n: (0, 0), memory_space=pltpu.SMEM),
        ],
        out_shape=[
            jax.ShapeDtypeStruct((_N, 2 * _CH, _H, _W), jnp.float32),
            jax.ShapeDtypeStruct((1, 2), jnp.float32),
        ],
    )(parts.reshape(8, 128), x, masks)

    denom = jnp.float32(_L * _H * _W)
    rate = (counts[0, 0] / denom + counts[0, 1] / denom) / 2
    return xo, rate
